# Initial kernel scaffold; baseline (speedup 1.0000x reference)
import jax
import jax.numpy as jnp
from jax.experimental import pallas as pl


def kernel(x, W_conf, b_conf, wealth, base_gate, base_up, base_down, gate_A, gate_B, up_A, up_B, down_A, down_B):
    raise NotImplementedError("write your pallas kernel here")


# speedup vs baseline: 1.8418x; 1.8418x over previous
import functools

import jax
import jax.numpy as jnp
from jax import lax
from jax.experimental import pallas as pl
from jax.experimental.pallas import tpu as pltpu
from jax.experimental.pallas import tpu_sc as plsc

E, K, D, I, R = 8, 2, 768, 2048, 64
SCALING = 16.0 / 64.0
T = 2048
BT = 256
ER = E * R


def _bids_body(x_ref, wct_ref, b_ref, wealth_ref, out_ref):
    logits = jnp.dot(x_ref[...], wct_ref[...], preferred_element_type=jnp.float32)
    logits = logits + b_ref[...]
    out_ref[...] = jax.nn.sigmoid(logits) * wealth_ref[...]


def _compute_bids(x2, W_conf, b_conf, wealth):
    return pl.pallas_call(
        _bids_body,
        out_shape=jax.ShapeDtypeStruct((T, E), jnp.float32),
    )(x2, W_conf.T, b_conf.reshape(1, E), wealth.reshape(1, E))


def _route(bids):
    info = plsc.get_sparse_core_info()
    nc, ns = info.num_cores, info.num_subcores
    nw = nc * ns
    per = T // nw
    chunk = per * E

    @functools.partial(
        pl.kernel,
        out_type=jax.ShapeDtypeStruct((T * E,), jnp.float32),
        mesh=plsc.VectorSubcoreMesh(core_axis_name="c", subcore_axis_name="s"),
        scratch_types=[
            pltpu.VMEM((chunk,), jnp.float32),
            pltpu.VMEM((chunk,), jnp.float32),
        ],
    )
    def k(bids_hbm, out_hbm, bv, cv):
        wid = lax.axis_index("s") * nc + lax.axis_index("c")
        base = wid * chunk
        pltpu.sync_copy(bids_hbm.at[pl.ds(base, chunk)], bv)
        for g in range(per // 16):
            b = [bv[pl.ds(e * per + g * 16, 16)] for e in range(E)]
            m1 = b[0]
            a1 = jnp.zeros((16,), jnp.int32)
            m2 = jnp.full((16,), -jnp.inf, jnp.float32)
            a2 = jnp.full((16,), -1, jnp.int32)
            for e in range(1, E):
                be = b[e]
                gt1 = be > m1
                gt2 = be > m2
                m2n = jnp.where(gt1, m1, jnp.where(gt2, be, m2))
                a2n = jnp.where(gt1, a1, jnp.where(gt2, e, a2))
                m1 = jnp.where(gt1, be, m1)
                a1 = jnp.where(gt1, e, a1)
                m2, a2 = m2n, a2n
            t = jnp.exp(m2 - m1)
            denom = 1.0 + t
            w1 = 1.0 / denom
            w2 = t / denom
            for e in range(E):
                ce = jnp.where(a1 == e, w1, jnp.where(a2 == e, w2, 0.0))
                cv[pl.ds(e * per + g * 16, 16)] = ce
        pltpu.sync_copy(cv, out_hbm.at[pl.ds(base, chunk)])

    bw = bids.reshape(nw, per, E).transpose(0, 2, 1).reshape(nw * E * per)
    out = k(bw)
    return out.reshape(nw, E, per).transpose(0, 2, 1).reshape(T, E)


def _ffn_body(x_ref, c_ref, A_ref, gB_ref, uB_ref, bg_ref, bu_ref, bd_ref,
              dA_ref, dB_ref, y_ref):
    xb = x_ref[...]
    g0 = jnp.dot(xb, bg_ref[...], preferred_element_type=jnp.float32)
    u0 = jnp.dot(xb, bu_ref[...], preferred_element_type=jnp.float32)
    P = jnp.dot(xb, A_ref[...], preferred_element_type=jnp.float32) * SCALING
    Pb = P.astype(jnp.bfloat16)
    gB = gB_ref[...]
    uB = uB_ref[...]
    dA = dA_ref[...]
    c = c_ref[...]
    accH = jnp.zeros((BT, I), jnp.float32)
    qs = []
    for e in range(E):
        pg = Pb[:, e * R:(e + 1) * R]
        pu = Pb[:, ER + e * R:ER + (e + 1) * R]
        dg = jnp.dot(pg, gB[e * R:(e + 1) * R, :], preferred_element_type=jnp.float32)
        du = jnp.dot(pu, uB[e * R:(e + 1) * R, :], preferred_element_type=jnp.float32)
        g = g0 + dg
        u = u0 + du
        h = (g * jax.nn.sigmoid(g)) * u
        ch = h * c[:, e:e + 1]
        accH = accH + ch
        q = jnp.dot(ch.astype(jnp.bfloat16), dA[:, e * R:(e + 1) * R],
                    preferred_element_type=jnp.float32)
        qs.append(q)
    Q = jnp.concatenate(qs, axis=1).astype(jnp.bfloat16)
    y = jnp.dot(accH.astype(jnp.bfloat16), bd_ref[...], preferred_element_type=jnp.float32)
    y = y + SCALING * jnp.dot(Q, dB_ref[...], preferred_element_type=jnp.float32)
    y_ref[...] = y


def _ffn(xb16, c, A_all, gB_all, uB_all, bg, bu, bd, dA_all, dB_all):
    full = lambda shape: pl.BlockSpec(shape, lambda i: (0, 0))
    return pl.pallas_call(
        _ffn_body,
        grid=(T // BT,),
        in_specs=[
            pl.BlockSpec((BT, D), lambda i: (i, 0)),
            pl.BlockSpec((BT, E), lambda i: (i, 0)),
            full((D, 2 * ER)),
            full((ER, I)),
            full((ER, I)),
            full((D, I)),
            full((D, I)),
            full((I, D)),
            full((I, ER)),
            full((ER, D)),
        ],
        out_specs=pl.BlockSpec((BT, D), lambda i: (i, 0)),
        out_shape=jax.ShapeDtypeStruct((T, D), jnp.float32),
        compiler_params=pltpu.CompilerParams(
            dimension_semantics=("arbitrary",),
        ),
    )(xb16, c, A_all, gB_all, uB_all, bg, bu, bd, dA_all, dB_all)


def kernel(x, W_conf, b_conf, wealth, base_gate, base_up, base_down,
           gate_A, gate_B, up_A, up_B, down_A, down_B):
    Bx, Sx, Dx = x.shape
    x2 = x.reshape(T, D)

    bids = _compute_bids(x2, W_conf, b_conf, wealth)
    combine = _route(bids)

    bf = jnp.bfloat16
    xb16 = x2.astype(bf)
    A_all = jnp.concatenate(
        [gate_A.transpose(1, 0, 2).reshape(D, ER),
         up_A.transpose(1, 0, 2).reshape(D, ER)], axis=1).astype(bf)
    gB_all = gate_B.reshape(ER, I).astype(bf)
    uB_all = up_B.reshape(ER, I).astype(bf)
    dA_all = down_A.transpose(1, 0, 2).reshape(I, ER).astype(bf)
    dB_all = down_B.reshape(ER, D).astype(bf)

    y = _ffn(xb16, combine, A_all, gB_all, uB_all,
             base_gate.astype(bf), base_up.astype(bf), base_down.astype(bf),
             dA_all, dB_all)
    return y.reshape(Bx, Sx, Dx)


# speedup vs baseline: 2.0391x; 1.1071x over previous
import functools

import jax
import jax.numpy as jnp
from jax import lax
from jax.experimental import pallas as pl
from jax.experimental.pallas import tpu as pltpu
from jax.experimental.pallas import tpu_sc as plsc

E, K, D, I, R = 8, 2, 768, 2048, 64
SCALING = 16.0 / 64.0
T = 2048
BT = 256
ER = E * R


def _bids_body(x_ref, wct_ref, b_ref, wealth_ref, out_ref):
    logits = jnp.dot(x_ref[...], wct_ref[...], preferred_element_type=jnp.float32)
    logits = logits + b_ref[...]
    out_ref[...] = jax.nn.sigmoid(logits) * wealth_ref[...]


def _compute_bids(x2, W_conf, b_conf, wealth):
    return pl.pallas_call(
        _bids_body,
        out_shape=jax.ShapeDtypeStruct((T, E), jnp.float32),
    )(x2, W_conf.T, b_conf.reshape(1, E), wealth.reshape(1, E))


def _route(bids):
    info = plsc.get_sparse_core_info()
    nc, ns = info.num_cores, info.num_subcores
    nw = nc * ns
    per = T // nw
    chunk = per * E

    @functools.partial(
        pl.kernel,
        out_type=jax.ShapeDtypeStruct((T * E,), jnp.float32),
        mesh=plsc.VectorSubcoreMesh(core_axis_name="c", subcore_axis_name="s"),
        scratch_types=[
            pltpu.VMEM((chunk,), jnp.float32),
            pltpu.VMEM((chunk,), jnp.float32),
        ],
    )
    def k(bids_hbm, out_hbm, bv, cv):
        wid = lax.axis_index("s") * nc + lax.axis_index("c")
        base = wid * chunk
        pltpu.sync_copy(bids_hbm.at[pl.ds(base, chunk)], bv)
        for g in range(per // 16):
            b = [bv[pl.ds(e * per + g * 16, 16)] for e in range(E)]
            m1 = b[0]
            a1 = jnp.zeros((16,), jnp.int32)
            m2 = jnp.full((16,), -jnp.inf, jnp.float32)
            a2 = jnp.full((16,), -1, jnp.int32)
            for e in range(1, E):
                be = b[e]
                gt1 = be > m1
                gt2 = be > m2
                m2n = jnp.where(gt1, m1, jnp.where(gt2, be, m2))
                a2n = jnp.where(gt1, a1, jnp.where(gt2, e, a2))
                m1 = jnp.where(gt1, be, m1)
                a1 = jnp.where(gt1, e, a1)
                m2, a2 = m2n, a2n
            t = jnp.exp(m2 - m1)
            denom = 1.0 + t
            w1 = 1.0 / denom
            w2 = t / denom
            for e in range(E):
                ce = jnp.where(a1 == e, w1, jnp.where(a2 == e, w2, 0.0))
                cv[pl.ds(e * per + g * 16, 16)] = ce
        pltpu.sync_copy(cv, out_hbm.at[pl.ds(base, chunk)])

    bw = bids.reshape(nw, per, E).transpose(0, 2, 1).reshape(nw * E * per)
    out = k(bw)
    return out.reshape(nw, E, per).transpose(0, 2, 1).reshape(T, E)


def _ffn_body(x_ref, c_ref, A_ref, gB_ref, uB_ref, bg_ref, bu_ref, bd_ref,
              dA_ref, dB_ref, y_ref):
    bf = jnp.bfloat16
    xb = x_ref[...]
    g0 = jnp.dot(xb, bg_ref[...], preferred_element_type=jnp.float32).astype(bf)
    u0 = jnp.dot(xb, bu_ref[...], preferred_element_type=jnp.float32).astype(bf)
    P = jnp.dot(xb, A_ref[...], preferred_element_type=jnp.float32) * SCALING
    Pb = P.astype(bf)
    gB = gB_ref[...]
    uB = uB_ref[...]
    dA = dA_ref[...]
    c = c_ref[...]
    accH = jnp.zeros((BT, I), bf)
    qs = []
    for e in range(E):
        pg = Pb[:, e * R:(e + 1) * R]
        pu = Pb[:, ER + e * R:ER + (e + 1) * R]
        dg = jnp.dot(pg, gB[e * R:(e + 1) * R, :], preferred_element_type=jnp.float32)
        du = jnp.dot(pu, uB[e * R:(e + 1) * R, :], preferred_element_type=jnp.float32)
        g = g0.astype(jnp.float32) + dg
        u = u0.astype(jnp.float32) + du
        h = (g * jax.nn.sigmoid(g)) * u
        ch = (h * c[:, e:e + 1]).astype(bf)
        accH = accH + ch
        q = jnp.dot(ch, dA[:, e * R:(e + 1) * R],
                    preferred_element_type=jnp.float32).astype(bf)
        qs.append(q)
    Q = jnp.concatenate(qs, axis=1)
    y = jnp.dot(accH, bd_ref[...], preferred_element_type=jnp.float32)
    y = y + SCALING * jnp.dot(Q, dB_ref[...], preferred_element_type=jnp.float32)
    y_ref[...] = y


def _ffn(xb16, c, A_all, gB_all, uB_all, bg, bu, bd, dA_all, dB_all):
    full = lambda shape: pl.BlockSpec(shape, lambda i: (0, 0))
    return pl.pallas_call(
        _ffn_body,
        grid=(T // BT,),
        in_specs=[
            pl.BlockSpec((BT, D), lambda i: (i, 0)),
            pl.BlockSpec((BT, E), lambda i: (i, 0)),
            full((D, 2 * ER)),
            full((ER, I)),
            full((ER, I)),
            full((D, I)),
            full((D, I)),
            full((I, D)),
            full((I, ER)),
            full((ER, D)),
        ],
        out_specs=pl.BlockSpec((BT, D), lambda i: (i, 0)),
        out_shape=jax.ShapeDtypeStruct((T, D), jnp.float32),
        compiler_params=pltpu.CompilerParams(
            dimension_semantics=("arbitrary",),
        ),
    )(xb16, c, A_all, gB_all, uB_all, bg, bu, bd, dA_all, dB_all)


def kernel(x, W_conf, b_conf, wealth, base_gate, base_up, base_down,
           gate_A, gate_B, up_A, up_B, down_A, down_B):
    Bx, Sx, Dx = x.shape
    x2 = x.reshape(T, D)

    bids = _compute_bids(x2, W_conf, b_conf, wealth)
    combine = _route(bids)

    bf = jnp.bfloat16
    xb16 = x2.astype(bf)
    A_all = jnp.concatenate(
        [gate_A.transpose(1, 0, 2).reshape(D, ER),
         up_A.transpose(1, 0, 2).reshape(D, ER)], axis=1).astype(bf)
    gB_all = gate_B.reshape(ER, I).astype(bf)
    uB_all = up_B.reshape(ER, I).astype(bf)
    dA_all = down_A.transpose(1, 0, 2).reshape(I, ER).astype(bf)
    dB_all = down_B.reshape(ER, D).astype(bf)

    y = _ffn(xb16, combine, A_all, gB_all, uB_all,
             base_gate.astype(bf), base_up.astype(bf), base_down.astype(bf),
             dA_all, dB_all)
    return y.reshape(Bx, Sx, Dx)


# speedup vs baseline: 2.1723x; 1.0653x over previous
import functools

import jax
import jax.numpy as jnp
from jax import lax
from jax.experimental import pallas as pl
from jax.experimental.pallas import tpu as pltpu
from jax.experimental.pallas import tpu_sc as plsc

E, K, D, I, R = 8, 2, 768, 2048, 64
SCALING = 16.0 / 64.0
T = 2048
BT = 512
ER = E * R


def _bids_body(x_ref, wct_ref, b_ref, wealth_ref, out_ref, xb_ref):
    x = x_ref[...]
    logits = jnp.dot(x, wct_ref[...], preferred_element_type=jnp.float32)
    logits = logits + b_ref[...]
    out_ref[...] = jax.nn.sigmoid(logits) * wealth_ref[...]
    xb_ref[...] = x.astype(jnp.bfloat16)


def _compute_bids(x2, W_conf, b_conf, wealth):
    return pl.pallas_call(
        _bids_body,
        out_shape=[jax.ShapeDtypeStruct((T, E), jnp.float32),
                   jax.ShapeDtypeStruct((T, D), jnp.bfloat16)],
    )(x2, W_conf.T, b_conf.reshape(1, E), wealth.reshape(1, E))


def _route(bids):
    info = plsc.get_sparse_core_info()
    nc, ns = info.num_cores, info.num_subcores
    nw = nc * ns
    per = T // nw
    chunk = per * E

    @functools.partial(
        pl.kernel,
        out_type=jax.ShapeDtypeStruct((T * E,), jnp.float32),
        mesh=plsc.VectorSubcoreMesh(core_axis_name="c", subcore_axis_name="s"),
        scratch_types=[
            pltpu.VMEM((chunk,), jnp.float32),
            pltpu.VMEM((chunk,), jnp.float32),
        ],
    )
    def k(bids_hbm, out_hbm, bv, cv):
        wid = lax.axis_index("s") * nc + lax.axis_index("c")
        base = wid * chunk
        pltpu.sync_copy(bids_hbm.at[pl.ds(base, chunk)], bv)
        for g in range(per // 16):
            b = [bv[pl.ds(e * per + g * 16, 16)] for e in range(E)]
            m1 = b[0]
            a1 = jnp.zeros((16,), jnp.int32)
            m2 = jnp.full((16,), -jnp.inf, jnp.float32)
            a2 = jnp.full((16,), -1, jnp.int32)
            for e in range(1, E):
                be = b[e]
                gt1 = be > m1
                gt2 = be > m2
                m2n = jnp.where(gt1, m1, jnp.where(gt2, be, m2))
                a2n = jnp.where(gt1, a1, jnp.where(gt2, e, a2))
                m1 = jnp.where(gt1, be, m1)
                a1 = jnp.where(gt1, e, a1)
                m2, a2 = m2n, a2n
            t = jnp.exp(m2 - m1)
            denom = 1.0 + t
            w1 = 1.0 / denom
            w2 = t / denom
            for e in range(E):
                ce = jnp.where(a1 == e, w1, jnp.where(a2 == e, w2, 0.0))
                cv[pl.ds(e * per + g * 16, 16)] = ce
        pltpu.sync_copy(cv, out_hbm.at[pl.ds(base, chunk)])

    bw = bids.reshape(nw, per, E).transpose(0, 2, 1).reshape(nw * E * per)
    out = k(bw)
    return out.reshape(nw, E, per).transpose(0, 2, 1).reshape(T, E)


def _prep_body(bg_ref, bu_ref, bd_ref, gB_ref, uB_ref, dB_ref,
               bg_o, bu_o, bd_o, gB_o, uB_o, dB_o):
    bf = jnp.bfloat16
    bg_o[...] = bg_ref[...].astype(bf)
    bu_o[...] = bu_ref[...].astype(bf)
    bd_o[...] = bd_ref[...].astype(bf)
    gB_o[...] = gB_ref[...].astype(bf)
    uB_o[...] = uB_ref[...].astype(bf)
    dB_o[...] = dB_ref[...].astype(bf)


def _prep(base_gate, base_up, base_down, gB2, uB2, dB2):
    bf = jnp.bfloat16
    return pl.pallas_call(
        _prep_body,
        out_shape=[
            jax.ShapeDtypeStruct((D, I), bf),
            jax.ShapeDtypeStruct((D, I), bf),
            jax.ShapeDtypeStruct((I, D), bf),
            jax.ShapeDtypeStruct((ER, I), bf),
            jax.ShapeDtypeStruct((ER, I), bf),
            jax.ShapeDtypeStruct((ER, D), bf),
        ],
    )(base_gate, base_up, base_down, gB2, uB2, dB2)


def _ffn_body(x_ref, c_ref, A_ref, gB_ref, uB_ref, bg_ref, bu_ref, bd_ref,
              dA_ref, dB_ref, y_ref):
    bf = jnp.bfloat16
    xb = x_ref[...]
    g0 = jnp.dot(xb, bg_ref[...], preferred_element_type=jnp.float32).astype(bf)
    u0 = jnp.dot(xb, bu_ref[...], preferred_element_type=jnp.float32).astype(bf)
    P = jnp.dot(xb, A_ref[...], preferred_element_type=jnp.float32) * SCALING
    Pb = P.astype(bf)
    gB = gB_ref[...]
    uB = uB_ref[...]
    dA = dA_ref[...]
    c = c_ref[...]
    accH = jnp.zeros((BT, I), bf)
    qs = []
    for e in range(E):
        pg = Pb[:, e * R:(e + 1) * R]
        pu = Pb[:, ER + e * R:ER + (e + 1) * R]
        dg = jnp.dot(pg, gB[e * R:(e + 1) * R, :], preferred_element_type=jnp.float32)
        du = jnp.dot(pu, uB[e * R:(e + 1) * R, :], preferred_element_type=jnp.float32)
        g = g0.astype(jnp.float32) + dg
        u = u0.astype(jnp.float32) + du
        h = (g * jax.nn.sigmoid(g)) * u
        ch = (h * c[:, e:e + 1]).astype(bf)
        accH = accH + ch
        q = jnp.dot(ch, dA[:, e * R:(e + 1) * R],
                    preferred_element_type=jnp.float32).astype(bf)
        qs.append(q)
    Q = jnp.concatenate(qs, axis=1)
    y = jnp.dot(accH, bd_ref[...], preferred_element_type=jnp.float32)
    y = y + SCALING * jnp.dot(Q, dB_ref[...], preferred_element_type=jnp.float32)
    y_ref[...] = y


def _ffn(xb16, c, A_all, gB_all, uB_all, bg, bu, bd, dA_all, dB_all):
    full = lambda shape: pl.BlockSpec(shape, lambda i: (0, 0))
    return pl.pallas_call(
        _ffn_body,
        grid=(T // BT,),
        in_specs=[
            pl.BlockSpec((BT, D), lambda i: (i, 0)),
            pl.BlockSpec((BT, E), lambda i: (i, 0)),
            full((D, 2 * ER)),
            full((ER, I)),
            full((ER, I)),
            full((D, I)),
            full((D, I)),
            full((I, D)),
            full((I, ER)),
            full((ER, D)),
        ],
        out_specs=pl.BlockSpec((BT, D), lambda i: (i, 0)),
        out_shape=jax.ShapeDtypeStruct((T, D), jnp.float32),
        compiler_params=pltpu.CompilerParams(
            dimension_semantics=("arbitrary",),
        ),
    )(xb16, c, A_all, gB_all, uB_all, bg, bu, bd, dA_all, dB_all)


def kernel(x, W_conf, b_conf, wealth, base_gate, base_up, base_down,
           gate_A, gate_B, up_A, up_B, down_A, down_B):
    Bx, Sx, Dx = x.shape
    x2 = x.reshape(T, D)

    bids, xb16 = _compute_bids(x2, W_conf, b_conf, wealth)
    combine = _route(bids)

    bf = jnp.bfloat16
    A_all = jnp.concatenate(
        [gate_A.transpose(1, 0, 2).reshape(D, ER),
         up_A.transpose(1, 0, 2).reshape(D, ER)], axis=1).astype(bf)
    dA_all = down_A.transpose(1, 0, 2).reshape(I, ER).astype(bf)
    bg16, bu16, bd16, gB_all, uB_all, dB_all = _prep(
        base_gate, base_up, base_down,
        gate_B.reshape(ER, I), up_B.reshape(ER, I), down_B.reshape(ER, D))

    y = _ffn(xb16, combine, A_all, gB_all, uB_all,
             bg16, bu16, bd16, dA_all, dB_all)
    return y.reshape(Bx, Sx, Dx)
